# Initial kernel scaffold; baseline (speedup 1.0000x reference)
import jax
import jax.numpy as jnp
from jax.experimental import pallas as pl


def kernel(indices, offsets, weight):
    raise NotImplementedError("write your pallas kernel here")


# speedup vs baseline: 50.4262x; 50.4262x over previous
import jax
import jax.numpy as jnp
from jax import lax
from jax.experimental import pallas as pl
from jax.experimental.pallas import tpu as pltpu
from jax.experimental.pallas import tpu_sc as plsc

VOCAB = 1000000
DIM = 64
BATCH = 16384
TOTAL = 327680

NC = 2
NS = 16
NW = NC * NS
BPW = BATCH // NW
CH = 512
NSUB = CH // 128
OFF_PAD = 1032
ACC_ROWS = BPW + 8
DUMMY = BPW


def _body(off_hbm, idx_hbm, w_hbm, out_hbm,
          off_v, idx_b, seg_b, rows_v, outb_v, inv_v, acc_sh, sem):
    sid = lax.axis_index("s")
    wid = sid * NC + lax.axis_index("c")
    bag0 = pl.multiple_of(wid * BPW, 8)
    abase = sid * ACC_ROWS

    pltpu.sync_copy(off_hbm.at[pl.ds(bag0, OFF_PAD)], off_v)

    def _zero(r, _):
        for k in range(DIM // 16):
            outb_v[r, pl.ds(k * 16, 16)] = jnp.zeros((16,), jnp.float32)
        return 0
    lax.fori_loop(0, BPW, _zero, 0)
    pltpu.sync_copy(outb_v, acc_sh.at[pl.ds(abase, BPW)])
    pltpu.sync_copy(outb_v.at[pl.ds(0, 8)],
                    acc_sh.at[pl.ds(abase + BPW, 8)])

    t0 = off_v[pl.ds(0, 16)][0]
    t1 = off_v[pl.ds(BPW, 16)][0]
    c0a = pl.multiple_of(lax.bitwise_and(t0, jnp.int32(-8)), 8)
    span = t1 - c0a
    nch = lax.div(span + (CH - 1), jnp.int32(CH))

    lane = lax.iota(jnp.int32, 16)

    def _chunk(i, _):
        c0 = pl.multiple_of(c0a + i * CH, 8)
        for j in range(NSUB):
            pltpu.sync_copy(idx_hbm.at[pl.ds(c0 + 128 * j, 128)], idx_b[j])
        descs = [pltpu.async_copy(w_hbm.at[idx_b[j]],
                                  rows_v.at[pl.ds(128 * j, 128)], sem)
                 for j in range(NSUB)]
        for j in range(NSUB):
            for q in range(128 // 16):
                p = c0 + 128 * j + 16 * q + lane
                c = jnp.zeros((16,), jnp.int32)
                for s in (512, 256, 128, 64, 32, 16, 8, 4, 2, 1):
                    nc2 = c + s
                    val = plsc.load_gather(off_v, [nc2 - 1])
                    c = jnp.where(val <= p, nc2, c)
                seg = jnp.where(c == 0, DUMMY, jnp.minimum(c - 1, DUMMY))
                seg_b[j][pl.ds(16 * q, 16)] = seg + abase
        for d in descs:
            d.wait()
        for j in range(NSUB):
            pltpu.sync_copy(rows_v.at[pl.ds(128 * j, 128)],
                            acc_sh.at[seg_b[j]], add=True)
        return 0

    lax.fori_loop(0, nch, _chunk, 0)

    for g in range(BPW // 16):
        a = plsc.load_gather(off_v, [lane + g * 16])
        b = plsc.load_gather(off_v, [lane + g * 16 + 1])
        cnt = (b - a).astype(jnp.float32)
        inv_v[pl.ds(g * 16, 16)] = 1.0 / jnp.maximum(cnt, 1.0)

    pltpu.sync_copy(acc_sh.at[pl.ds(abase, BPW)], outb_v)

    def _scale(r, _):
        s = inv_v[pl.ds(r, 16)][0]
        for k in range(DIM // 16):
            outb_v[r, pl.ds(k * 16, 16)] = outb_v[r, pl.ds(k * 16, 16)] * s
        return 0
    lax.fori_loop(0, BPW, _scale, 0)

    pltpu.sync_copy(outb_v, out_hbm.at[pl.ds(bag0, BPW)])


@jax.jit
def _run(offsets_ext, indices_pad, weight):
    mesh = plsc.VectorSubcoreMesh(core_axis_name="c", subcore_axis_name="s")
    scratch = (
        pltpu.VMEM((OFF_PAD,), jnp.int32),
        [pltpu.VMEM((128,), jnp.int32) for _ in range(NSUB)],
        [pltpu.VMEM((128,), jnp.int32) for _ in range(NSUB)],
        pltpu.VMEM((CH, DIM), jnp.float32),
        pltpu.VMEM((BPW, DIM), jnp.float32),
        pltpu.VMEM((BPW + 16,), jnp.float32),
        pltpu.VMEM_SHARED((NS * ACC_ROWS, DIM), jnp.float32),
        pltpu.SemaphoreType.DMA,
    )
    return pl.kernel(
        _body,
        out_type=jax.ShapeDtypeStruct((BATCH, DIM), jnp.float32),
        mesh=mesh,
        scratch_types=scratch,
        compiler_params=pltpu.CompilerParams(
            needs_layout_passes=False, use_tc_tiling_on_sc=False),
    )(offsets_ext, indices_pad, weight)


def kernel(indices, offsets, weight):
    offsets_ext = jnp.concatenate(
        [offsets, jnp.full((OFF_PAD,), TOTAL, jnp.int32)])
    indices_pad = jnp.concatenate(
        [indices, jnp.zeros((CH,), jnp.int32)])
    return _run(offsets_ext, indices_pad, weight)


# speedup vs baseline: 50.5445x; 1.0023x over previous
import jax
import jax.numpy as jnp
from jax import lax
from jax.experimental.layout import Format, Layout
from jax.experimental import pallas as pl
from jax.experimental.pallas import tpu as pltpu
from jax.experimental.pallas import tpu_sc as plsc

VOCAB = 1000000
DIM = 64
BATCH = 16384
TOTAL = 327680

NC = 2
NS = 16
NW = NC * NS
BPW = BATCH // NW
CH = 512
NSUB = CH // 128
OFF_PAD = 1032
ACC_ROWS = BPW + 8
DUMMY = BPW


def _body(off_hbm, idx_hbm, w_hbm, out_hbm,
          off_v, idx_b, seg_b, rows_v, outb_v, inv_v, acc_sh, sem):
    sid = lax.axis_index("s")
    wid = sid * NC + lax.axis_index("c")
    bag0 = pl.multiple_of(wid * BPW, 8)
    abase = sid * ACC_ROWS

    pltpu.sync_copy(off_hbm.at[pl.ds(bag0, OFF_PAD)], off_v)

    def _zero(r, _):
        for k in range(DIM // 16):
            outb_v[r, pl.ds(k * 16, 16)] = jnp.zeros((16,), jnp.float32)
        return 0
    lax.fori_loop(0, BPW, _zero, 0)
    pltpu.sync_copy(outb_v, acc_sh.at[pl.ds(abase, BPW)])
    pltpu.sync_copy(outb_v.at[pl.ds(0, 8)],
                    acc_sh.at[pl.ds(abase + BPW, 8)])

    t0 = off_v[pl.ds(0, 16)][0]
    t1 = off_v[pl.ds(BPW, 16)][0]
    c0a = pl.multiple_of(lax.bitwise_and(t0, jnp.int32(-8)), 8)
    span = t1 - c0a
    nch = lax.div(span + (CH - 1), jnp.int32(CH))

    lane = lax.iota(jnp.int32, 16)

    def _chunk(i, _):
        c0 = pl.multiple_of(c0a + i * CH, 8)
        for j in range(NSUB):
            pltpu.sync_copy(idx_hbm.at[pl.ds(c0 + 128 * j, 128)], idx_b[j])
        descs = [pltpu.async_copy(w_hbm.at[idx_b[j]],
                                  rows_v.at[pl.ds(128 * j, 128)], sem)
                 for j in range(NSUB)]
        for j in range(NSUB):
            for q in range(128 // 16):
                p = c0 + 128 * j + 16 * q + lane
                c = jnp.zeros((16,), jnp.int32)
                for s in (512, 256, 128, 64, 32, 16, 8, 4, 2, 1):
                    nc2 = c + s
                    val = plsc.load_gather(off_v, [nc2 - 1])
                    c = jnp.where(val <= p, nc2, c)
                seg = jnp.where(c == 0, DUMMY, jnp.minimum(c - 1, DUMMY))
                seg_b[j][pl.ds(16 * q, 16)] = seg + abase
        for d in descs:
            d.wait()
        for j in range(NSUB):
            pltpu.sync_copy(rows_v.at[pl.ds(128 * j, 128)],
                            acc_sh.at[seg_b[j]], add=True)
        return 0

    lax.fori_loop(0, nch, _chunk, 0)

    for g in range(BPW // 16):
        a = plsc.load_gather(off_v, [lane + g * 16])
        b = plsc.load_gather(off_v, [lane + g * 16 + 1])
        cnt = (b - a).astype(jnp.float32)
        inv_v[pl.ds(g * 16, 16)] = 1.0 / jnp.maximum(cnt, 1.0)

    pltpu.sync_copy(acc_sh.at[pl.ds(abase, BPW)], outb_v)

    def _scale(r, _):
        s = inv_v[pl.ds(r, 16)][0]
        for k in range(DIM // 16):
            outb_v[r, pl.ds(k * 16, 16)] = outb_v[r, pl.ds(k * 16, 16)] * s
        return 0
    lax.fori_loop(0, BPW, _scale, 0)

    pltpu.sync_copy(outb_v, out_hbm.at[pl.ds(bag0, BPW)])


@jax.jit
def _run(offsets_ext, indices_pad, weight):
    mesh = plsc.VectorSubcoreMesh(core_axis_name="c", subcore_axis_name="s")
    scratch = (
        pltpu.VMEM((OFF_PAD,), jnp.int32),
        [pltpu.VMEM((128,), jnp.int32) for _ in range(NSUB)],
        [pltpu.VMEM((128,), jnp.int32) for _ in range(NSUB)],
        pltpu.VMEM((CH, DIM), jnp.float32),
        pltpu.VMEM((BPW, DIM), jnp.float32),
        pltpu.VMEM((BPW + 16,), jnp.float32),
        pltpu.VMEM_SHARED((NS * ACC_ROWS, DIM), jnp.float32),
        pltpu.SemaphoreType.DMA,
    )
    return pl.kernel(
        _body,
        out_type=jax.ShapeDtypeStruct((BATCH, DIM), jnp.float32),
        mesh=mesh,
        scratch_types=scratch,
        compiler_params=pltpu.CompilerParams(
            needs_layout_passes=False, use_tc_tiling_on_sc=False),
    )(offsets_ext, indices_pad, weight)


def kernel(indices, offsets, weight):
    sharding = jax.sharding.SingleDeviceSharding(jax.devices()[0])
    weight = jax.device_put(
        weight,
        Format(Layout(major_to_minor=(0, 1), tiling=((8,),)), sharding))
    offsets_ext = jnp.concatenate(
        [offsets, jnp.full((OFF_PAD,), TOTAL, jnp.int32)])
    indices_pad = jnp.concatenate(
        [indices, jnp.zeros((CH,), jnp.int32)])
    return _run(offsets_ext, indices_pad, weight)
